# baseline (speedup 1.0000x reference)
from functools import partial

import jax
import jax.numpy as jnp
from jax.experimental import pallas as pl

_BK = 4096


def _mm_kernel(nk: int, valid_last: int, x_ref, et_ref, o_ref):
    k = pl.program_id(0)

    @pl.when(k == 0)
    def _():
        o_ref[...] = jnp.zeros_like(o_ref)

    @pl.when(k < nk - 1)
    def _():
        o_ref[...] += jax.lax.dot_general(
            x_ref[...], et_ref[...],
            (((1,), (1,)), ((), ())),
            preferred_element_type=jnp.float32)

    @pl.when(k == nk - 1)
    def _():
        bk = x_ref.shape[1]
        col = jax.lax.broadcasted_iota(jnp.int32, (1, bk), 1)
        mask = col < valid_last
        x = jnp.where(mask, x_ref[...], 0.0)
        et = jnp.where(mask, et_ref[...], 0.0)
        o_ref[...] += jax.lax.dot_general(
            x, et, (((1,), (1,)), ((), ())),
            preferred_element_type=jnp.float32)


def kernel(inputs, embeddings):
    m, kdim = inputs.shape
    n = embeddings.shape[1]
    nk = (kdim + _BK - 1) // _BK
    valid_last = kdim - (nk - 1) * _BK

    emb_t = embeddings.T

    return pl.pallas_call(
        partial(_mm_kernel, nk, valid_last),
        grid=(nk,),
        in_specs=[
            pl.BlockSpec((m, _BK), lambda k: (0, k)),
            pl.BlockSpec((n, _BK), lambda k: (0, k)),
        ],
        out_specs=pl.BlockSpec((m, n), lambda k: (0, 0)),
        out_shape=jax.ShapeDtypeStruct((m, n), jnp.float32),
    )(inputs, emb_t)
